# baseline (speedup 1.0000x reference)
import functools

import jax
import jax.numpy as jnp
from jax import lax
from jax.experimental import pallas as pl
from jax.experimental.pallas import tpu as pltpu
from jax.experimental.pallas import tpu_sc as plsc

_B = 4
_T = 4096
_D = 768
_N = _B * _T
_NC = 2
_NS = 16
_NW = _NC * _NS
_P_W = _T // _NW
_CK = 32
_KPB = _P_W // _CK
_LANES = 16
_GRP = _D // _LANES

_SCHED = [(q, b) for q in range(_KPB) for b in range(_B)]


def _make_emb_kernel():
    mesh = plsc.VectorSubcoreMesh(core_axis_name="c", subcore_axis_name="s")

    @functools.partial(
        pl.kernel,
        out_type=jax.ShapeDtypeStruct((_N, _D), jnp.float32),
        mesh=mesh,
        scratch_types=[
            pltpu.VMEM((_B * _KPB, _CK), jnp.int32),
            pltpu.VMEM((_CK, _D), jnp.float32),
            pltpu.VMEM((_CK, _D), jnp.float32),
            pltpu.VMEM((_CK, _D), jnp.float32),
            pltpu.VMEM((_CK, _D), jnp.float32),
            pltpu.VMEM((_CK, _D), jnp.float32),
            pltpu.SemaphoreType.DMA,
            pltpu.SemaphoreType.DMA,
            pltpu.SemaphoreType.DMA,
        ],
    )
    def emb(xw_hbm, tok_hbm, pos_hbm, out_hbm,
            idx_v, posa_v, posb_v, tok0_v, tok1_v, tok2_v, psem, gsem, ssem):
        wid = lax.axis_index("s") * _NC + lax.axis_index("c")
        p0 = pl.multiple_of(wid * _P_W, _P_W)
        bufs = (tok0_v, tok1_v, tok2_v)
        pos_bufs = (posa_v, posb_v)
        n = len(_SCHED)

        pltpu.sync_copy(xw_hbm.at[wid], idx_v)

        def load_pos(q):
            rows = pl.multiple_of(p0 + q * _CK, _CK)
            return pltpu.async_copy(pos_hbm.at[pl.ds(rows, _CK)],
                                    pos_bufs[q % 2], psem)

        def gather(ci):
            q, b = _SCHED[ci]
            return pltpu.async_copy(tok_hbm.at[idx_v.at[b * _KPB + q]],
                                    bufs[ci % 3], gsem)

        def store(ci):
            q, b = _SCHED[ci]
            rows = pl.multiple_of(b * _T + p0 + q * _CK, _CK)
            return pltpu.async_copy(bufs[ci % 3],
                                    out_hbm.at[pl.ds(rows, _CK)], ssem)

        pos_cp = load_pos(0)
        g_cp = {0: gather(0)}
        s_cp = {}
        for ci, (q, b) in enumerate(_SCHED):
            buf = bufs[ci % 3]
            pos_v = pos_bufs[q % 2]
            g_cp.pop(ci).wait()
            if ci - 2 in s_cp:
                s_cp.pop(ci - 2).wait()
            if ci + 1 < n:
                g_cp[ci + 1] = gather(ci + 1)
            if b == 0:
                pos_cp.wait()
                if q + 1 < _KPB:
                    pos_cp = load_pos(q + 1)

            @plsc.parallel_loop(0, _CK, unroll=2)
            def _(i, buf=buf, pos_v=pos_v):
                for g in range(_GRP):
                    s = pl.ds(g * _LANES, _LANES)
                    plsc.addupdate(buf.at[i, s], pos_v[i, s])

            s_cp[ci] = store(ci)
        s_cp.pop(n - 2).wait()
        s_cp.pop(n - 1).wait()

    return emb


_emb = _make_emb_kernel()


def kernel(x, token_table, pos_table):
    xw = (x.reshape(_B, _NW, _P_W)
           .transpose(1, 0, 2)
           .reshape(_NW, _B * _KPB, _CK)
           .astype(jnp.int32))
    out = _emb(xw, token_table, pos_table)
    return out.reshape(_B, _T, _D)


# speedup vs baseline: 1.0299x; 1.0299x over previous
import functools

import jax
import jax.numpy as jnp
from jax import lax
from jax.experimental import pallas as pl
from jax.experimental.pallas import tpu as pltpu
from jax.experimental.pallas import tpu_sc as plsc

_B = 4
_T = 4096
_D = 768
_N = _B * _T
_NC = 2
_NS = 16
_NW = _NC * _NS
_P_W = _T // _NW
_CK = 32
_KPB = _P_W // _CK
_LANES = 16
_GRP = _D // _LANES

_SCHED = [(q, b) for q in range(_KPB) for b in range(_B)]


def _make_emb_kernel():
    mesh = plsc.VectorSubcoreMesh(core_axis_name="c", subcore_axis_name="s")

    @functools.partial(
        pl.kernel,
        out_type=jax.ShapeDtypeStruct((_N, _D), jnp.float32),
        mesh=mesh,
        scratch_types=[
            pltpu.VMEM((_B * _KPB, _CK), jnp.int32),
            pltpu.VMEM((_CK, _D), jnp.float32),
            pltpu.VMEM((_CK, _D), jnp.float32),
            pltpu.VMEM((_CK, _D), jnp.float32),
            pltpu.VMEM((_CK, _D), jnp.float32),
            pltpu.VMEM((_CK, _D), jnp.float32),
            pltpu.SemaphoreType.DMA,
            pltpu.SemaphoreType.DMA,
            pltpu.SemaphoreType.DMA,
        ],
    )
    def emb(xw_hbm, tok_hbm, pos_hbm, out_hbm,
            idx_v, posa_v, posb_v, tok0_v, tok1_v, tok2_v, psem, gsem, ssem):
        wid = lax.axis_index("s") * _NC + lax.axis_index("c")
        p0 = pl.multiple_of(wid * _P_W, _P_W)
        bufs = (tok0_v, tok1_v, tok2_v)
        pos_bufs = (posa_v, posb_v)
        n = len(_SCHED)

        pltpu.sync_copy(xw_hbm.at[wid], idx_v)

        def load_pos(q):
            rows = pl.multiple_of(p0 + q * _CK, _CK)
            return pltpu.async_copy(pos_hbm.at[pl.ds(rows, _CK)],
                                    pos_bufs[q % 2], psem)

        def gather(ci):
            q, b = _SCHED[ci]
            return pltpu.async_copy(tok_hbm.at[idx_v.at[b * _KPB + q]],
                                    bufs[ci % 3], gsem)

        def store(ci):
            q, b = _SCHED[ci]
            rows = pl.multiple_of(b * _T + p0 + q * _CK, _CK)
            return pltpu.async_copy(bufs[ci % 3],
                                    out_hbm.at[pl.ds(rows, _CK)], ssem)

        pos_cp = load_pos(0)
        g_cp = {0: gather(0), 1: gather(1)}
        s_cp = {}
        for ci, (q, b) in enumerate(_SCHED):
            buf = bufs[ci % 3]
            pos_v = pos_bufs[q % 2]
            g_cp.pop(ci).wait()
            if ci - 2 in s_cp:
                s_cp.pop(ci - 2).wait()
            if ci + 1 < n and ci + 1 not in g_cp:
                g_cp[ci + 1] = gather(ci + 1)
            if b == 0:
                pos_cp.wait()
                if q + 1 < _KPB:
                    pos_cp = load_pos(q + 1)

            @plsc.parallel_loop(0, _CK, unroll=1)
            def _(i, buf=buf, pos_v=pos_v):
                for g in range(_GRP):
                    s = pl.ds(g * _LANES, _LANES)
                    plsc.addupdate(buf.at[i, s], pos_v[i, s])

            s_cp[ci] = store(ci)
        s_cp.pop(n - 2).wait()
        s_cp.pop(n - 1).wait()

    return emb


_emb = _make_emb_kernel()


def kernel(x, token_table, pos_table):
    xw = (x.reshape(_B, _NW, _P_W)
           .transpose(1, 0, 2)
           .reshape(_NW, _B * _KPB, _CK)
           .astype(jnp.int32))
    out = _emb(xw, token_table, pos_table)
    return out.reshape(_B, _T, _D)


# speedup vs baseline: 1.0496x; 1.0191x over previous
import functools

import jax
import jax.numpy as jnp
from jax import lax
from jax.experimental import pallas as pl
from jax.experimental.pallas import tpu as pltpu
from jax.experimental.pallas import tpu_sc as plsc

_B = 4
_T = 4096
_D = 768
_N = _B * _T
_NC = 2
_NS = 16
_NW = _NC * _NS
_P_W = _T // _NW
_CK = 32
_KPB = _P_W // _CK
_LANES = 16
_GRP = _D // _LANES

_SCHED = [(q, b) for q in range(_KPB) for b in range(_B)]


def _make_emb_kernel():
    mesh = plsc.VectorSubcoreMesh(core_axis_name="c", subcore_axis_name="s")

    @functools.partial(
        pl.kernel,
        out_type=jax.ShapeDtypeStruct((_N, _D), jnp.float32),
        mesh=mesh,
        scratch_types=[
            pltpu.VMEM((_B * _KPB, _CK), jnp.int32),
            pltpu.VMEM((_CK, _D), jnp.float32),
            pltpu.VMEM((_CK, _D), jnp.float32),
            pltpu.VMEM((_CK, _D), jnp.float32),
            pltpu.VMEM((_CK, _D), jnp.float32),
            pltpu.VMEM((_CK, _D), jnp.float32),
            pltpu.SemaphoreType.DMA,
            pltpu.SemaphoreType.DMA,
            pltpu.SemaphoreType.DMA,
        ],
    )
    def emb(xw_hbm, tok_hbm, pos_hbm, out_hbm,
            idx_v, posa_v, posb_v, tok0_v, tok1_v, tok2_v, psem, gsem, ssem):
        wid = lax.axis_index("s") * _NC + lax.axis_index("c")
        p0 = pl.multiple_of(wid * _P_W, _P_W)
        bufs = (tok0_v, tok1_v, tok2_v)
        pos_bufs = (posa_v, posb_v)
        n = len(_SCHED)

        pltpu.sync_copy(xw_hbm.at[wid], idx_v)

        def load_pos(q):
            rows = pl.multiple_of(p0 + q * _CK, _CK)
            return pltpu.async_copy(pos_hbm.at[pl.ds(rows, _CK)],
                                    pos_bufs[q % 2], psem)

        def gather(ci):
            q, b = _SCHED[ci]
            return pltpu.async_copy(tok_hbm.at[idx_v.at[b * _KPB + q]],
                                    bufs[ci % 3], gsem)

        def store(ci):
            q, b = _SCHED[ci]
            rows = pl.multiple_of(b * _T + p0 + q * _CK, _CK)
            return pltpu.async_copy(bufs[ci % 3],
                                    out_hbm.at[pl.ds(rows, _CK)], ssem)

        pos_cp = load_pos(0)
        g_cp = {0: gather(0), 1: gather(1)}
        s_cp = {}
        for ci, (q, b) in enumerate(_SCHED):
            buf = bufs[ci % 3]
            pos_v = pos_bufs[q % 2]
            g_cp.pop(ci).wait()
            if ci - 2 in s_cp:
                s_cp.pop(ci - 2).wait()
            if ci + 1 < n and ci + 1 not in g_cp:
                g_cp[ci + 1] = gather(ci + 1)
            if b == 0:
                pos_cp.wait()
                if q + 1 < _KPB:
                    pos_cp = load_pos(q + 1)

            @plsc.parallel_loop(0, _GRP, unroll=1)
            def _(g, buf=buf, pos_v=pos_v):
                s = pl.ds(g * _LANES, _LANES)
                for i in range(_CK):
                    plsc.addupdate(buf.at[i, s], pos_v[i, s])

            s_cp[ci] = store(ci)
        s_cp.pop(n - 2).wait()
        s_cp.pop(n - 1).wait()

    return emb


_emb = _make_emb_kernel()


def kernel(x, token_table, pos_table):
    xw = (x.reshape(_B, _NW, _P_W)
           .transpose(1, 0, 2)
           .reshape(_NW, _B * _KPB, _CK)
           .astype(jnp.int32))
    out = _emb(xw, token_table, pos_table)
    return out.reshape(_B, _T, _D)
